# Initial kernel scaffold; baseline (speedup 1.0000x reference)
import jax
import jax.numpy as jnp
from jax.experimental import pallas as pl


def kernel(x, edge, edge_weight, parameter, bias1, p1, p2, agg_bias, pred_w, pred_b):
    raise NotImplementedError("write your pallas kernel here")


# speedup vs baseline: 78.0019x; 78.0019x over previous
import functools

import jax
import jax.numpy as jnp
from jax import lax
from jax.experimental import pallas as pl
from jax.experimental.pallas import tpu as pltpu
from jax.experimental.pallas import tpu_sc as plsc

N_NODES = 100000
N_EDGES = 3200000
DIM_IN = 128
BATCH = 4
N_GCN = 3

NC = 2
NS = 16
NPAD = 100096
NPT = NPAD // NS
EPT = N_EDGES // NS
CHUNK = 4000
NCHUNK = EPT // CHUNK
NVEC = NPT // 16


def _proj_body(x_ref, w_ref, b_ref, o_ref):
    x = x_ref[...].astype(jnp.bfloat16).astype(jnp.float32)
    w = w_ref[...].astype(jnp.bfloat16).astype(jnp.float32)
    y = jnp.sum(x * w[0][None, None, :], axis=-1) + b_ref[0, 0]
    o_ref[...] = y[None]


def _project(x, w, bias1):
    BN = 1000
    grid = N_NODES // BN
    out = pl.pallas_call(
        _proj_body,
        grid=(grid,),
        in_specs=[
            pl.BlockSpec((BATCH, BN, DIM_IN), lambda i: (0, i, 0)),
            pl.BlockSpec((1, DIM_IN), lambda i: (0, 0)),
            pl.BlockSpec(memory_space=pltpu.SMEM),
        ],
        out_specs=pl.BlockSpec((1, BATCH, BN), lambda i: (i, 0, 0)),
        out_shape=jax.ShapeDtypeStruct((grid, BATCH, BN), jnp.float32),
    )(x, w, bias1)
    return out.transpose(1, 0, 2).reshape(BATCH, N_NODES)


def _bf16_round(v):
    bits = plsc.bitcast(v, jnp.int32)
    lsb = lax.bitwise_and(lax.shift_right_logical(bits, 16), 1)
    r = lax.bitwise_and(bits + (lsb + 0x7FFF), -65536)
    return plsc.bitcast(r, jnp.float32)


def _sc_body(h0_hbm, src_hbm, dst_hbm, pw_hbm, par_hbm, ones_hbm, zeros_hbm,
             out_hbm,
             t0, t1, s0a, s1a, dg,
             srcb, dstb, gat0, gat1, onesb,
             sl0, sl1, dsl, h0b, h1b, zb, pwb, parb, ovec,
             sem, sem2):
    c = lax.axis_index("c")
    s = lax.axis_index("s")
    nb = s * NPT
    eb = s * EPT
    nsl = pl.ds(nb, NPT)

    iota = lax.iota(jnp.int32, 16)

    pltpu.sync_copy(par_hbm, parb)
    pltpu.sync_copy(ones_hbm, onesb)
    pltpu.sync_copy(zeros_hbm, zb)
    pltpu.sync_copy(pw_hbm.at[nsl], pwb)
    pltpu.sync_copy(h0_hbm.at[c, 0, nsl], h0b)
    pltpu.sync_copy(h0_hbm.at[c, 1, nsl], h1b)
    pltpu.sync_copy(h0b, t0.at[nsl])
    pltpu.sync_copy(h1b, t1.at[nsl])
    pltpu.sync_copy(zb, s0a.at[nsl])
    pltpu.sync_copy(zb, s1a.at[nsl])
    pltpu.sync_copy(zb, dg.at[nsl])
    plsc.subcore_barrier()

    accs = (jnp.zeros((16,), jnp.float32), jnp.zeros((16,), jnp.float32))

    for k in range(N_GCN):
        def edge_step(i, carry):
            base = eb + i * CHUNK
            pltpu.sync_copy(src_hbm.at[pl.ds(base, CHUNK)], srcb)
            pltpu.sync_copy(dst_hbm.at[pl.ds(base, CHUNK)], dstb)
            pltpu.async_copy(t0.at[srcb], gat0, sem).wait()
            pltpu.async_copy(t1.at[srcb], gat1, sem).wait()
            pltpu.async_copy(gat0, s0a.at[dstb], sem2, add=True).wait()
            pltpu.async_copy(gat1, s1a.at[dstb], sem2, add=True).wait()
            if k == 0:
                pltpu.async_copy(onesb, dg.at[dstb], sem2, add=True).wait()
            return carry

        lax.fori_loop(0, NCHUNK, edge_step, 0)
        plsc.subcore_barrier()

        pltpu.sync_copy(s0a.at[nsl], sl0)
        pltpu.sync_copy(s1a.at[nsl], sl1)
        if k == 0:
            pltpu.sync_copy(dg.at[nsl], dsl)

        ak = parb.at[k][pl.ds(0, 16)]
        bk = parb.at[k][pl.ds(16, 16)]
        ck = parb.at[k][pl.ds(32, 16)]

        def node_step(j, carry):
            a0, a1 = carry
            d16 = pl.ds(j * 16, 16)
            dv = dsl[d16]
            denom = jnp.maximum(dv, 1.0)
            hn0 = (ak * sl0[d16] + bk * dv * h0b[d16]) / denom + ck
            hn0 = jnp.where(hn0 >= 0.0, hn0, 0.1 * hn0)
            h0b[d16] = hn0
            hn1 = (ak * sl1[d16] + bk * dv * h1b[d16]) / denom + ck
            hn1 = jnp.where(hn1 >= 0.0, hn1, 0.1 * hn1)
            h1b[d16] = hn1
            if k == N_GCN - 1:
                pwv = pwb[d16]
                a0 = a0 + _bf16_round(hn0) * pwv
                a1 = a1 + _bf16_round(hn1) * pwv
            return (a0, a1)

        accs = lax.fori_loop(0, NVEC, node_step, accs)

        if k < N_GCN - 1:
            pltpu.sync_copy(zb, s0a.at[nsl])
            pltpu.sync_copy(zb, s1a.at[nsl])
            pltpu.sync_copy(h0b, t0.at[nsl])
            pltpu.sync_copy(h1b, t1.at[nsl])
            plsc.subcore_barrier()

    p0 = jnp.sum(accs[0])
    p1 = jnp.sum(accs[1])
    res = jnp.where(iota == 0, p0, jnp.where(iota == 1, p1, 0.0))
    ovec[...] = res
    pltpu.sync_copy(ovec, out_hbm.at[c, s])


_sc_kernel = functools.partial(
    pl.kernel,
    out_type=jax.ShapeDtypeStruct((NC, NS, 16), jnp.float32),
    mesh=plsc.VectorSubcoreMesh(core_axis_name="c", subcore_axis_name="s"),
    compiler_params=pltpu.CompilerParams(
        use_tc_tiling_on_sc=False, needs_layout_passes=False
    ),
    scratch_types=[
        pltpu.VMEM_SHARED((NPAD,), jnp.float32),
        pltpu.VMEM_SHARED((NPAD,), jnp.float32),
        pltpu.VMEM_SHARED((NPAD,), jnp.float32),
        pltpu.VMEM_SHARED((NPAD,), jnp.float32),
        pltpu.VMEM_SHARED((NPAD,), jnp.float32),
        pltpu.VMEM((CHUNK,), jnp.int32),
        pltpu.VMEM((CHUNK,), jnp.int32),
        pltpu.VMEM((CHUNK,), jnp.float32),
        pltpu.VMEM((CHUNK,), jnp.float32),
        pltpu.VMEM((CHUNK,), jnp.float32),
        pltpu.VMEM((NPT,), jnp.float32),
        pltpu.VMEM((NPT,), jnp.float32),
        pltpu.VMEM((NPT,), jnp.float32),
        pltpu.VMEM((NPT,), jnp.float32),
        pltpu.VMEM((NPT,), jnp.float32),
        pltpu.VMEM((NPT,), jnp.float32),
        pltpu.VMEM((NPT,), jnp.float32),
        pltpu.VMEM((N_GCN, 48), jnp.float32),
        pltpu.VMEM((16,), jnp.float32),
        pltpu.SemaphoreType.DMA,
        pltpu.SemaphoreType.DMA,
    ],
)(_sc_body)


@jax.jit
def kernel(x, edge, edge_weight, parameter, bias1, p1, p2, agg_bias,
           pred_w, pred_b):
    del edge_weight
    w = (jnp.abs(parameter) / DIM_IN).reshape(1, DIM_IN)
    h0 = _project(x, w, bias1)
    h0p = jnp.pad(h0, ((0, 0), (0, NPAD - N_NODES)))
    h0s = h0p.reshape(NC, 2, NPAD)

    src = edge[0]
    dst = edge[1]

    lane = jnp.ones((16,), jnp.float32)
    a3 = jnp.abs(p1[:, 0, 0])
    b3 = jnp.abs(p2[:, 0, 0])
    c3 = agg_bias[:, 0, 0]
    par = jnp.concatenate(
        [a3[:, None] * lane, b3[:, None] * lane, c3[:, None] * lane], axis=1
    )

    pw_r = pred_w[0].astype(jnp.bfloat16).astype(jnp.float32)
    pw = jnp.pad(pw_r, (0, NPAD - N_NODES))
    ones_c = jnp.ones((CHUNK,), jnp.float32)
    zeros_c = jnp.zeros((NPT,), jnp.float32)

    parts = _sc_kernel(h0s, src, dst, pw, par, ones_c, zeros_c)
    out = parts[:, :, :2].sum(axis=1).reshape(BATCH, 1) + pred_b[None, :]
    return out


# speedup vs baseline: 82.3119x; 1.0553x over previous
import functools

import jax
import jax.numpy as jnp
from jax import lax
from jax.experimental import pallas as pl
from jax.experimental.pallas import tpu as pltpu
from jax.experimental.pallas import tpu_sc as plsc

N_NODES = 100000
N_EDGES = 3200000
DIM_IN = 128
BATCH = 4
N_GCN = 3

NC = 2
NS = 16
NPAD = 100096
NPT = NPAD // NS
EPT = N_EDGES // NS
CHUNK = 4000
NCHUNK = EPT // CHUNK
NVEC = NPT // 16


def _proj_body(x_ref, w_ref, b_ref, o_ref):
    x = x_ref[...].astype(jnp.bfloat16).astype(jnp.float32)
    w = w_ref[...].astype(jnp.bfloat16).astype(jnp.float32)
    y = jnp.sum(x * w[0][None, None, :], axis=-1) + b_ref[0, 0]
    o_ref[...] = y[None]


def _project(x, w, bias1):
    BN = 1000
    grid = N_NODES // BN
    out = pl.pallas_call(
        _proj_body,
        grid=(grid,),
        in_specs=[
            pl.BlockSpec((BATCH, BN, DIM_IN), lambda i: (0, i, 0)),
            pl.BlockSpec((1, DIM_IN), lambda i: (0, 0)),
            pl.BlockSpec(memory_space=pltpu.SMEM),
        ],
        out_specs=pl.BlockSpec((1, BATCH, BN), lambda i: (i, 0, 0)),
        out_shape=jax.ShapeDtypeStruct((grid, BATCH, BN), jnp.float32),
    )(x, w, bias1)
    return out.transpose(1, 0, 2).reshape(BATCH, N_NODES)


def _bf16_round(v):
    bits = plsc.bitcast(v, jnp.int32)
    lsb = lax.bitwise_and(lax.shift_right_logical(bits, 16), 1)
    r = lax.bitwise_and(bits + (lsb + 0x7FFF), -65536)
    return plsc.bitcast(r, jnp.float32)


def _sc_body(h0_hbm, src_hbm, dst_hbm, pw_hbm, par_hbm, ones_hbm, zeros_hbm,
             out_hbm,
             t0, t1, s0a, s1a, dg,
             srcb, dstb, gat0, gat1, onesb,
             sl0, sl1, dsl, h0b, h1b, zb, pwb, parb, ovec,
             sem_i, sem_g, sem_s):
    c = lax.axis_index("c")
    s = lax.axis_index("s")
    nb = s * NPT
    eb = s * EPT
    nsl = pl.ds(nb, NPT)

    iota = lax.iota(jnp.int32, 16)

    pltpu.sync_copy(par_hbm, parb)
    pltpu.sync_copy(ones_hbm, onesb)
    pltpu.sync_copy(zeros_hbm, zb)
    pltpu.sync_copy(pw_hbm.at[nsl], pwb)
    pltpu.sync_copy(h0_hbm.at[c, 0, nsl], h0b)
    pltpu.sync_copy(h0_hbm.at[c, 1, nsl], h1b)
    pltpu.sync_copy(h0b, t0.at[nsl])
    pltpu.sync_copy(h1b, t1.at[nsl])
    pltpu.sync_copy(zb, s0a.at[nsl])
    pltpu.sync_copy(zb, s1a.at[nsl])
    pltpu.sync_copy(zb, dg.at[nsl])
    plsc.subcore_barrier()

    accs = (jnp.zeros((16,), jnp.float32), jnp.zeros((16,), jnp.float32))

    for k in range(N_GCN):
        def issue_idx(j, p):
            base = eb + j * CHUNK
            pltpu.async_copy(src_hbm.at[pl.ds(base, CHUNK)], srcb.at[p], sem_i)
            pltpu.async_copy(dst_hbm.at[pl.ds(base, CHUNK)], dstb.at[p], sem_i)

        def wait_idx(j, p):
            base = eb + j * CHUNK
            pltpu.make_async_copy(
                src_hbm.at[pl.ds(base, CHUNK)], srcb.at[p], sem_i).wait()
            pltpu.make_async_copy(
                dst_hbm.at[pl.ds(base, CHUNK)], dstb.at[p], sem_i).wait()

        def issue_scat(p):
            pltpu.async_copy(gat0.at[p], s0a.at[dstb.at[p]], sem_s, add=True)
            pltpu.async_copy(gat1.at[p], s1a.at[dstb.at[p]], sem_s, add=True)
            if k == 0:
                pltpu.async_copy(onesb, dg.at[dstb.at[p]], sem_s, add=True)

        def wait_scat(p):
            pltpu.make_async_copy(gat0.at[p], s0a.at[dstb.at[p]], sem_s).wait()
            pltpu.make_async_copy(gat1.at[p], s1a.at[dstb.at[p]], sem_s).wait()
            if k == 0:
                pltpu.make_async_copy(onesb, dg.at[dstb.at[p]], sem_s).wait()

        issue_idx(0, 0)

        def edge_pair(i, carry):
            for p in (0, 1):
                j = 2 * i + p
                wait_idx(j, p)
                g0 = pltpu.async_copy(t0.at[srcb.at[p]], gat0.at[p], sem_g)
                g1 = pltpu.async_copy(t1.at[srcb.at[p]], gat1.at[p], sem_g)
                if p == 0:
                    @pl.when(i > 0)
                    def _():
                        wait_scat(1)
                    issue_idx(j + 1, 1)
                else:
                    wait_scat(0)

                    @pl.when(i < NCHUNK // 2 - 1)
                    def _():
                        issue_idx(j + 1, 0)
                g0.wait()
                g1.wait()
                issue_scat(p)
            return carry

        lax.fori_loop(0, NCHUNK // 2, edge_pair, 0)
        wait_scat(1)
        plsc.subcore_barrier()

        pltpu.sync_copy(s0a.at[nsl], sl0)
        pltpu.sync_copy(s1a.at[nsl], sl1)
        if k == 0:
            pltpu.sync_copy(dg.at[nsl], dsl)

        ak = parb.at[k][pl.ds(0, 16)]
        bk = parb.at[k][pl.ds(16, 16)]
        ck = parb.at[k][pl.ds(32, 16)]

        def node_step(j, carry):
            a0, a1 = carry
            d16 = pl.ds(j * 16, 16)
            dv = dsl[d16]
            denom = jnp.maximum(dv, 1.0)
            hn0 = (ak * sl0[d16] + bk * dv * h0b[d16]) / denom + ck
            hn0 = jnp.where(hn0 >= 0.0, hn0, 0.1 * hn0)
            h0b[d16] = hn0
            hn1 = (ak * sl1[d16] + bk * dv * h1b[d16]) / denom + ck
            hn1 = jnp.where(hn1 >= 0.0, hn1, 0.1 * hn1)
            h1b[d16] = hn1
            if k == N_GCN - 1:
                pwv = pwb[d16]
                a0 = a0 + _bf16_round(hn0) * pwv
                a1 = a1 + _bf16_round(hn1) * pwv
            return (a0, a1)

        accs = lax.fori_loop(0, NVEC, node_step, accs)

        if k < N_GCN - 1:
            pltpu.sync_copy(zb, s0a.at[nsl])
            pltpu.sync_copy(zb, s1a.at[nsl])
            pltpu.sync_copy(h0b, t0.at[nsl])
            pltpu.sync_copy(h1b, t1.at[nsl])
            plsc.subcore_barrier()

    p0 = jnp.sum(accs[0])
    p1 = jnp.sum(accs[1])
    res = jnp.where(iota == 0, p0, jnp.where(iota == 1, p1, 0.0))
    ovec[...] = res
    pltpu.sync_copy(ovec, out_hbm.at[c, s])


_sc_kernel = functools.partial(
    pl.kernel,
    out_type=jax.ShapeDtypeStruct((NC, NS, 16), jnp.float32),
    mesh=plsc.VectorSubcoreMesh(core_axis_name="c", subcore_axis_name="s"),
    compiler_params=pltpu.CompilerParams(
        use_tc_tiling_on_sc=False, needs_layout_passes=False
    ),
    scratch_types=[
        pltpu.VMEM_SHARED((NPAD,), jnp.float32),
        pltpu.VMEM_SHARED((NPAD,), jnp.float32),
        pltpu.VMEM_SHARED((NPAD,), jnp.float32),
        pltpu.VMEM_SHARED((NPAD,), jnp.float32),
        pltpu.VMEM_SHARED((NPAD,), jnp.float32),
        pltpu.VMEM((2, CHUNK), jnp.int32),
        pltpu.VMEM((2, CHUNK), jnp.int32),
        pltpu.VMEM((2, CHUNK), jnp.float32),
        pltpu.VMEM((2, CHUNK), jnp.float32),
        pltpu.VMEM((CHUNK,), jnp.float32),
        pltpu.VMEM((NPT,), jnp.float32),
        pltpu.VMEM((NPT,), jnp.float32),
        pltpu.VMEM((NPT,), jnp.float32),
        pltpu.VMEM((NPT,), jnp.float32),
        pltpu.VMEM((NPT,), jnp.float32),
        pltpu.VMEM((NPT,), jnp.float32),
        pltpu.VMEM((NPT,), jnp.float32),
        pltpu.VMEM((N_GCN, 48), jnp.float32),
        pltpu.VMEM((16,), jnp.float32),
        pltpu.SemaphoreType.DMA,
        pltpu.SemaphoreType.DMA,
        pltpu.SemaphoreType.DMA,
    ],
)(_sc_body)


@jax.jit
def kernel(x, edge, edge_weight, parameter, bias1, p1, p2, agg_bias,
           pred_w, pred_b):
    del edge_weight
    w = (jnp.abs(parameter) / DIM_IN).reshape(1, DIM_IN)
    h0 = _project(x, w, bias1)
    h0p = jnp.pad(h0, ((0, 0), (0, NPAD - N_NODES)))
    h0s = h0p.reshape(NC, 2, NPAD)

    src = edge[0]
    dst = edge[1]

    lane = jnp.ones((16,), jnp.float32)
    a3 = jnp.abs(p1[:, 0, 0])
    b3 = jnp.abs(p2[:, 0, 0])
    c3 = agg_bias[:, 0, 0]
    par = jnp.concatenate(
        [a3[:, None] * lane, b3[:, None] * lane, c3[:, None] * lane], axis=1
    )

    pw_r = pred_w[0].astype(jnp.bfloat16).astype(jnp.float32)
    pw = jnp.pad(pw_r, (0, NPAD - N_NODES))
    ones_c = jnp.ones((CHUNK,), jnp.float32)
    zeros_c = jnp.zeros((NPT,), jnp.float32)

    parts = _sc_kernel(h0s, src, dst, pw, par, ones_c, zeros_c)
    out = parts[:, :, :2].sum(axis=1).reshape(BATCH, 1) + pred_b[None, :]
    return out


# speedup vs baseline: 88.0365x; 1.0695x over previous
import functools

import jax
import jax.numpy as jnp
from jax import lax
from jax.experimental import pallas as pl
from jax.experimental.pallas import tpu as pltpu
from jax.experimental.pallas import tpu_sc as plsc

N_NODES = 100000
N_EDGES = 3200000
DIM_IN = 128
BATCH = 4
N_GCN = 3

NC = 2
NS = 16
NPAD = 100096
NPT = NPAD // NS
EPT = N_EDGES // NS
CHUNK = 5000
NCHUNK = EPT // CHUNK
NVEC = NPT // 16


def _proj_body(x_ref, w_ref, b_ref, o_ref):
    x = x_ref[...].astype(jnp.bfloat16).astype(jnp.float32)
    w = w_ref[...].astype(jnp.bfloat16).astype(jnp.float32)
    y = jnp.sum(x * w[0][None, None, :], axis=-1) + b_ref[0, 0]
    o_ref[...] = y[None]


def _project(x, w, bias1):
    BN = 1000
    grid = N_NODES // BN
    out = pl.pallas_call(
        _proj_body,
        grid=(grid,),
        in_specs=[
            pl.BlockSpec((BATCH, BN, DIM_IN), lambda i: (0, i, 0)),
            pl.BlockSpec((1, DIM_IN), lambda i: (0, 0)),
            pl.BlockSpec(memory_space=pltpu.SMEM),
        ],
        out_specs=pl.BlockSpec((1, BATCH, BN), lambda i: (i, 0, 0)),
        out_shape=jax.ShapeDtypeStruct((grid, BATCH, BN), jnp.float32),
    )(x, w, bias1)
    return out.transpose(1, 0, 2).reshape(BATCH, N_NODES)


def _bf16_round(v):
    bits = plsc.bitcast(v, jnp.int32)
    lsb = lax.bitwise_and(lax.shift_right_logical(bits, 16), 1)
    r = lax.bitwise_and(bits + (lsb + 0x7FFF), -65536)
    return plsc.bitcast(r, jnp.float32)


def _sc_body(h0_hbm, src_hbm, dst_hbm, pw_hbm, par_hbm, ones_hbm, zeros_hbm,
             out_hbm,
             t0, t1, s0a, s1a, dg,
             srcb, dstb, gat0, gat1, onesb,
             sl0, sl1, dsl, h0b, h1b, zb, pwb, parb, ovec,
             sem_i, sem_g, sem_s):
    c = lax.axis_index("c")
    s = lax.axis_index("s")
    nb = s * NPT
    eb = s * EPT
    nsl = pl.ds(nb, NPT)

    iota = lax.iota(jnp.int32, 16)

    pltpu.sync_copy(par_hbm, parb)
    pltpu.sync_copy(ones_hbm, onesb)
    pltpu.sync_copy(zeros_hbm, zb)
    pltpu.sync_copy(pw_hbm.at[nsl], pwb)
    pltpu.sync_copy(h0_hbm.at[c, 0, nsl], h0b)
    pltpu.sync_copy(h0_hbm.at[c, 1, nsl], h1b)
    pltpu.sync_copy(h0b, t0.at[nsl])
    pltpu.sync_copy(h1b, t1.at[nsl])
    pltpu.sync_copy(zb, s0a.at[nsl])
    pltpu.sync_copy(zb, s1a.at[nsl])
    pltpu.sync_copy(zb, dg.at[nsl])
    plsc.subcore_barrier()

    accs = (jnp.zeros((16,), jnp.float32), jnp.zeros((16,), jnp.float32))

    for k in range(N_GCN):
        def issue_idx(j, p):
            base = eb + j * CHUNK
            pltpu.async_copy(src_hbm.at[pl.ds(base, CHUNK)], srcb.at[p], sem_i)
            pltpu.async_copy(dst_hbm.at[pl.ds(base, CHUNK)], dstb.at[p], sem_i)

        def wait_idx(j, p):
            base = eb + j * CHUNK
            pltpu.make_async_copy(
                src_hbm.at[pl.ds(base, CHUNK)], srcb.at[p], sem_i).wait()
            pltpu.make_async_copy(
                dst_hbm.at[pl.ds(base, CHUNK)], dstb.at[p], sem_i).wait()

        def issue_scat(p):
            pltpu.async_copy(gat0.at[p], s0a.at[dstb.at[p]], sem_s, add=True)
            pltpu.async_copy(gat1.at[p], s1a.at[dstb.at[p]], sem_s, add=True)
            if k == 0:
                pltpu.async_copy(onesb, dg.at[dstb.at[p]], sem_s, add=True)

        def wait_scat(p):
            pltpu.make_async_copy(gat0.at[p], s0a.at[dstb.at[p]], sem_s).wait()
            pltpu.make_async_copy(gat1.at[p], s1a.at[dstb.at[p]], sem_s).wait()
            if k == 0:
                pltpu.make_async_copy(onesb, dg.at[dstb.at[p]], sem_s).wait()

        issue_idx(0, 0)

        def edge_pair(i, carry):
            for p in (0, 1):
                j = 2 * i + p
                wait_idx(j, p)
                g0 = pltpu.async_copy(t0.at[srcb.at[p]], gat0.at[p], sem_g)
                g1 = pltpu.async_copy(t1.at[srcb.at[p]], gat1.at[p], sem_g)
                if p == 0:
                    @pl.when(i > 0)
                    def _():
                        wait_scat(1)
                    issue_idx(j + 1, 1)
                else:
                    wait_scat(0)

                    @pl.when(i < NCHUNK // 2 - 1)
                    def _():
                        issue_idx(j + 1, 0)
                g0.wait()
                g1.wait()
                issue_scat(p)
            return carry

        lax.fori_loop(0, NCHUNK // 2, edge_pair, 0)
        wait_scat(1)
        plsc.subcore_barrier()

        pltpu.sync_copy(s0a.at[nsl], sl0)
        pltpu.sync_copy(s1a.at[nsl], sl1)
        if k == 0:
            pltpu.sync_copy(dg.at[nsl], dsl)

        ak = parb.at[k][pl.ds(0, 16)]
        bk = parb.at[k][pl.ds(16, 16)]
        ck = parb.at[k][pl.ds(32, 16)]

        def node_step(j, carry):
            a0, a1 = carry
            d16 = pl.ds(j * 16, 16)
            dv = dsl[d16]
            denom = jnp.maximum(dv, 1.0)
            hn0 = (ak * sl0[d16] + bk * dv * h0b[d16]) / denom + ck
            hn0 = jnp.where(hn0 >= 0.0, hn0, 0.1 * hn0)
            h0b[d16] = hn0
            hn1 = (ak * sl1[d16] + bk * dv * h1b[d16]) / denom + ck
            hn1 = jnp.where(hn1 >= 0.0, hn1, 0.1 * hn1)
            h1b[d16] = hn1
            if k == N_GCN - 1:
                pwv = pwb[d16]
                a0 = a0 + _bf16_round(hn0) * pwv
                a1 = a1 + _bf16_round(hn1) * pwv
            return (a0, a1)

        accs = lax.fori_loop(0, NVEC, node_step, accs)

        if k < N_GCN - 1:
            pltpu.sync_copy(zb, s0a.at[nsl])
            pltpu.sync_copy(zb, s1a.at[nsl])
            pltpu.sync_copy(h0b, t0.at[nsl])
            pltpu.sync_copy(h1b, t1.at[nsl])
            plsc.subcore_barrier()

    p0 = jnp.sum(accs[0])
    p1 = jnp.sum(accs[1])
    res = jnp.where(iota == 0, p0, jnp.where(iota == 1, p1, 0.0))
    ovec[...] = res
    pltpu.sync_copy(ovec, out_hbm.at[c, s])


_sc_kernel = functools.partial(
    pl.kernel,
    out_type=jax.ShapeDtypeStruct((NC, NS, 16), jnp.float32),
    mesh=plsc.VectorSubcoreMesh(core_axis_name="c", subcore_axis_name="s"),
    compiler_params=pltpu.CompilerParams(
        use_tc_tiling_on_sc=False, needs_layout_passes=False
    ),
    scratch_types=[
        pltpu.VMEM_SHARED((NPAD,), jnp.float32),
        pltpu.VMEM_SHARED((NPAD,), jnp.float32),
        pltpu.VMEM_SHARED((NPAD,), jnp.float32),
        pltpu.VMEM_SHARED((NPAD,), jnp.float32),
        pltpu.VMEM_SHARED((NPAD,), jnp.float32),
        pltpu.VMEM((2, CHUNK), jnp.int32),
        pltpu.VMEM((2, CHUNK), jnp.int32),
        pltpu.VMEM((2, CHUNK), jnp.float32),
        pltpu.VMEM((2, CHUNK), jnp.float32),
        pltpu.VMEM((CHUNK,), jnp.float32),
        pltpu.VMEM((NPT,), jnp.float32),
        pltpu.VMEM((NPT,), jnp.float32),
        pltpu.VMEM((NPT,), jnp.float32),
        pltpu.VMEM((NPT,), jnp.float32),
        pltpu.VMEM((NPT,), jnp.float32),
        pltpu.VMEM((NPT,), jnp.float32),
        pltpu.VMEM((NPT,), jnp.float32),
        pltpu.VMEM((N_GCN, 48), jnp.float32),
        pltpu.VMEM((16,), jnp.float32),
        pltpu.SemaphoreType.DMA,
        pltpu.SemaphoreType.DMA,
        pltpu.SemaphoreType.DMA,
    ],
)(_sc_body)


@jax.jit
def kernel(x, edge, edge_weight, parameter, bias1, p1, p2, agg_bias,
           pred_w, pred_b):
    del edge_weight
    w = (jnp.abs(parameter) / DIM_IN).reshape(1, DIM_IN)
    h0 = _project(x, w, bias1)
    h0p = jnp.pad(h0, ((0, 0), (0, NPAD - N_NODES)))
    h0s = h0p.reshape(NC, 2, NPAD)

    src = edge[0]
    dst = edge[1]

    lane = jnp.ones((16,), jnp.float32)
    a3 = jnp.abs(p1[:, 0, 0])
    b3 = jnp.abs(p2[:, 0, 0])
    c3 = agg_bias[:, 0, 0]
    par = jnp.concatenate(
        [a3[:, None] * lane, b3[:, None] * lane, c3[:, None] * lane], axis=1
    )

    pw_r = pred_w[0].astype(jnp.bfloat16).astype(jnp.float32)
    pw = jnp.pad(pw_r, (0, NPAD - N_NODES))
    ones_c = jnp.ones((CHUNK,), jnp.float32)
    zeros_c = jnp.zeros((NPT,), jnp.float32)

    parts = _sc_kernel(h0s, src, dst, pw, par, ones_c, zeros_c)
    out = parts[:, :, :2].sum(axis=1).reshape(BATCH, 1) + pred_b[None, :]
    return out


# speedup vs baseline: 99.8414x; 1.1341x over previous
import functools

import jax
import jax.numpy as jnp
from jax import lax
from jax.experimental import pallas as pl
from jax.experimental.pallas import tpu as pltpu
from jax.experimental.pallas import tpu_sc as plsc

N_NODES = 100000
N_EDGES = 3200000
DIM_IN = 128
BATCH = 4
N_GCN = 3

NC = 2
NS = 16
NPAD = 100096
NPT = NPAD // NS
EPT = N_EDGES // NS
CHUNK = 5000
NCHUNK = EPT // CHUNK
NVEC = NPT // 16


def _proj_body(x_ref, w_ref, b_ref, o_ref):
    x = x_ref[...].astype(jnp.bfloat16).astype(jnp.float32)
    w = w_ref[...].astype(jnp.bfloat16).astype(jnp.float32)
    y = jnp.sum(x * w[0][None, None, :], axis=-1) + b_ref[0, 0]
    o_ref[...] = y[None]


def _project(x, w, bias1):
    BN = 1000
    grid = N_NODES // BN
    out = pl.pallas_call(
        _proj_body,
        grid=(grid,),
        in_specs=[
            pl.BlockSpec((BATCH, BN, DIM_IN), lambda i: (0, i, 0)),
            pl.BlockSpec((1, DIM_IN), lambda i: (0, 0)),
            pl.BlockSpec(memory_space=pltpu.SMEM),
        ],
        out_specs=pl.BlockSpec((1, BATCH, BN), lambda i: (i, 0, 0)),
        out_shape=jax.ShapeDtypeStruct((grid, BATCH, BN), jnp.float32),
    )(x, w, bias1)
    return out.transpose(1, 0, 2).reshape(BATCH, N_NODES)


def _deg_body(dst_hbm, ones_hbm, zeros_hbm, deg_hbm, dg, dstb, onesb, zb,
              sem_i, sem_s):
    c = lax.axis_index("c")
    s = lax.axis_index("s")
    nb = s * NPT
    nsl = pl.ds(nb, NPT)
    eb = (s + c * NS) * (N_EDGES // (2 * NS) // CHUNK) * CHUNK
    ndeg = N_EDGES // (2 * NS) // CHUNK

    pltpu.sync_copy(ones_hbm, onesb)
    pltpu.sync_copy(zeros_hbm, zb)
    pltpu.sync_copy(zb, dg.at[nsl])
    plsc.subcore_barrier()

    def issue_idx(j, p):
        base = eb + j * CHUNK
        pltpu.async_copy(dst_hbm.at[pl.ds(base, CHUNK)], dstb.at[p], sem_i)

    def wait_idx(j, p):
        base = eb + j * CHUNK
        pltpu.make_async_copy(
            dst_hbm.at[pl.ds(base, CHUNK)], dstb.at[p], sem_i).wait()

    def wait_scat(p):
        pltpu.make_async_copy(onesb, dg.at[dstb.at[p]], sem_s).wait()

    issue_idx(0, 0)

    def pair(i, carry):
        for p in (0, 1):
            j = 2 * i + p
            wait_idx(j, p)
            if p == 0:
                @pl.when(i > 0)
                def _():
                    wait_scat(1)
                issue_idx(j + 1, 1)
            else:
                wait_scat(0)

                @pl.when(i < ndeg // 2 - 1)
                def _():
                    issue_idx(j + 1, 0)
            pltpu.async_copy(onesb, dg.at[dstb.at[p]], sem_s, add=True)
        return carry

    lax.fori_loop(0, ndeg // 2, pair, 0)
    wait_scat(1)
    plsc.subcore_barrier()

    pltpu.sync_copy(dg.at[nsl], zb)
    pltpu.sync_copy(zb, deg_hbm.at[c, nsl])


_deg_kernel = functools.partial(
    pl.kernel,
    out_type=jax.ShapeDtypeStruct((NC, NPAD), jnp.float32),
    mesh=plsc.VectorSubcoreMesh(core_axis_name="c", subcore_axis_name="s"),
    compiler_params=pltpu.CompilerParams(
        use_tc_tiling_on_sc=False, needs_layout_passes=False
    ),
    scratch_types=[
        pltpu.VMEM_SHARED((NPAD,), jnp.float32),
        pltpu.VMEM((2, CHUNK), jnp.int32),
        pltpu.VMEM((CHUNK,), jnp.float32),
        pltpu.VMEM((NPT,), jnp.float32),
        pltpu.SemaphoreType.DMA,
        pltpu.SemaphoreType.DMA,
    ],
)(_deg_body)


def _bf16_round(v):
    bits = plsc.bitcast(v, jnp.int32)
    lsb = lax.bitwise_and(lax.shift_right_logical(bits, 16), 1)
    r = lax.bitwise_and(bits + (lsb + 0x7FFF), -65536)
    return plsc.bitcast(r, jnp.float32)


def _sc_body(h0_hbm, src_hbm, dst_hbm, pw_hbm, par_hbm, deg_hbm, zeros_hbm,
             out_hbm,
             t0, t1, s0a, s1a,
             srcb, dstb, gat0, gat1,
             sl0, sl1, dsl, h0b, h1b, zb, pwb, parb, ovec,
             sem_i, sem_g, sem_s):
    c = lax.axis_index("c")
    s = lax.axis_index("s")
    nb = s * NPT
    eb = s * EPT
    nsl = pl.ds(nb, NPT)

    iota = lax.iota(jnp.int32, 16)

    pltpu.sync_copy(par_hbm, parb)
    pltpu.sync_copy(zeros_hbm, zb)
    pltpu.sync_copy(pw_hbm.at[nsl], pwb)
    pltpu.sync_copy(h0_hbm.at[c, 0, nsl], h0b)
    pltpu.sync_copy(h0_hbm.at[c, 1, nsl], h1b)
    pltpu.sync_copy(h0b, t0.at[nsl])
    pltpu.sync_copy(h1b, t1.at[nsl])
    pltpu.sync_copy(zb, s0a.at[nsl])
    pltpu.sync_copy(zb, s1a.at[nsl])
    pltpu.sync_copy(deg_hbm.at[0, nsl], dsl)
    pltpu.sync_copy(deg_hbm.at[1, nsl], sl0)

    def deg_sum(j, carry):
        d16 = pl.ds(j * 16, 16)
        dsl[d16] = dsl[d16] + sl0[d16]
        return carry

    lax.fori_loop(0, NVEC, deg_sum, 0)
    plsc.subcore_barrier()

    accs = (jnp.zeros((16,), jnp.float32), jnp.zeros((16,), jnp.float32))

    for k in range(N_GCN):
        def issue_idx(j, p):
            base = eb + j * CHUNK
            pltpu.async_copy(src_hbm.at[pl.ds(base, CHUNK)], srcb.at[p], sem_i)
            pltpu.async_copy(dst_hbm.at[pl.ds(base, CHUNK)], dstb.at[p], sem_i)

        def wait_idx(j, p):
            base = eb + j * CHUNK
            pltpu.make_async_copy(
                src_hbm.at[pl.ds(base, CHUNK)], srcb.at[p], sem_i).wait()
            pltpu.make_async_copy(
                dst_hbm.at[pl.ds(base, CHUNK)], dstb.at[p], sem_i).wait()

        def issue_scat(p):
            pltpu.async_copy(gat0.at[p], s0a.at[dstb.at[p]], sem_s, add=True)
            pltpu.async_copy(gat1.at[p], s1a.at[dstb.at[p]], sem_s, add=True)

        def wait_scat(p):
            pltpu.make_async_copy(gat0.at[p], s0a.at[dstb.at[p]], sem_s).wait()
            pltpu.make_async_copy(gat1.at[p], s1a.at[dstb.at[p]], sem_s).wait()

        issue_idx(0, 0)

        def edge_pair(i, carry):
            for p in (0, 1):
                j = 2 * i + p
                wait_idx(j, p)
                g0 = pltpu.async_copy(t0.at[srcb.at[p]], gat0.at[p], sem_g)
                g1 = pltpu.async_copy(t1.at[srcb.at[p]], gat1.at[p], sem_g)
                if p == 0:
                    @pl.when(i > 0)
                    def _():
                        wait_scat(1)
                    issue_idx(j + 1, 1)
                else:
                    wait_scat(0)

                    @pl.when(i < NCHUNK // 2 - 1)
                    def _():
                        issue_idx(j + 1, 0)
                g0.wait()
                g1.wait()
                issue_scat(p)
            return carry

        lax.fori_loop(0, NCHUNK // 2, edge_pair, 0)
        wait_scat(1)
        plsc.subcore_barrier()

        pltpu.sync_copy(s0a.at[nsl], sl0)
        pltpu.sync_copy(s1a.at[nsl], sl1)

        ak = parb.at[k][pl.ds(0, 16)]
        bk = parb.at[k][pl.ds(16, 16)]
        ck = parb.at[k][pl.ds(32, 16)]

        def node_step(j, carry):
            a0, a1 = carry
            d16 = pl.ds(j * 16, 16)
            dv = dsl[d16]
            denom = jnp.maximum(dv, 1.0)
            hn0 = (ak * sl0[d16] + bk * dv * h0b[d16]) / denom + ck
            hn0 = jnp.where(hn0 >= 0.0, hn0, 0.1 * hn0)
            h0b[d16] = hn0
            hn1 = (ak * sl1[d16] + bk * dv * h1b[d16]) / denom + ck
            hn1 = jnp.where(hn1 >= 0.0, hn1, 0.1 * hn1)
            h1b[d16] = hn1
            if k == N_GCN - 1:
                pwv = pwb[d16]
                a0 = a0 + _bf16_round(hn0) * pwv
                a1 = a1 + _bf16_round(hn1) * pwv
            return (a0, a1)

        accs = lax.fori_loop(0, NVEC, node_step, accs)

        if k < N_GCN - 1:
            pltpu.sync_copy(zb, s0a.at[nsl])
            pltpu.sync_copy(zb, s1a.at[nsl])
            pltpu.sync_copy(h0b, t0.at[nsl])
            pltpu.sync_copy(h1b, t1.at[nsl])
            plsc.subcore_barrier()

    p0 = jnp.sum(accs[0])
    p1 = jnp.sum(accs[1])
    res = jnp.where(iota == 0, p0, jnp.where(iota == 1, p1, 0.0))
    ovec[...] = res
    pltpu.sync_copy(ovec, out_hbm.at[c, s])


_sc_kernel = functools.partial(
    pl.kernel,
    out_type=jax.ShapeDtypeStruct((NC, NS, 16), jnp.float32),
    mesh=plsc.VectorSubcoreMesh(core_axis_name="c", subcore_axis_name="s"),
    compiler_params=pltpu.CompilerParams(
        use_tc_tiling_on_sc=False, needs_layout_passes=False
    ),
    scratch_types=[
        pltpu.VMEM_SHARED((NPAD,), jnp.float32),
        pltpu.VMEM_SHARED((NPAD,), jnp.float32),
        pltpu.VMEM_SHARED((NPAD,), jnp.float32),
        pltpu.VMEM_SHARED((NPAD,), jnp.float32),
        pltpu.VMEM((2, CHUNK), jnp.int32),
        pltpu.VMEM((2, CHUNK), jnp.int32),
        pltpu.VMEM((2, CHUNK), jnp.float32),
        pltpu.VMEM((2, CHUNK), jnp.float32),
        pltpu.VMEM((NPT,), jnp.float32),
        pltpu.VMEM((NPT,), jnp.float32),
        pltpu.VMEM((NPT,), jnp.float32),
        pltpu.VMEM((NPT,), jnp.float32),
        pltpu.VMEM((NPT,), jnp.float32),
        pltpu.VMEM((NPT,), jnp.float32),
        pltpu.VMEM((NPT,), jnp.float32),
        pltpu.VMEM((N_GCN, 48), jnp.float32),
        pltpu.VMEM((16,), jnp.float32),
        pltpu.SemaphoreType.DMA,
        pltpu.SemaphoreType.DMA,
        pltpu.SemaphoreType.DMA,
    ],
)(_sc_body)


@jax.jit
def kernel(x, edge, edge_weight, parameter, bias1, p1, p2, agg_bias,
           pred_w, pred_b):
    del edge_weight
    w = (jnp.abs(parameter) / DIM_IN).reshape(1, DIM_IN)
    h0 = _project(x, w, bias1)
    h0p = jnp.pad(h0, ((0, 0), (0, NPAD - N_NODES)))
    h0s = h0p.reshape(NC, 2, NPAD)

    src = edge[0]
    dst = edge[1]

    lane = jnp.ones((16,), jnp.float32)
    a3 = jnp.abs(p1[:, 0, 0])
    b3 = jnp.abs(p2[:, 0, 0])
    c3 = agg_bias[:, 0, 0]
    par = jnp.concatenate(
        [a3[:, None] * lane, b3[:, None] * lane, c3[:, None] * lane], axis=1
    )

    pw_r = pred_w[0].astype(jnp.bfloat16).astype(jnp.float32)
    pw = jnp.pad(pw_r, (0, NPAD - N_NODES))
    ones_c = jnp.ones((CHUNK,), jnp.float32)
    zeros_c = jnp.zeros((NPT,), jnp.float32)

    deg = _deg_kernel(dst, ones_c, zeros_c)
    parts = _sc_kernel(h0s, src, dst, pw, par, deg, zeros_c)
    out = parts[:, :, :2].sum(axis=1).reshape(BATCH, 1) + pred_b[None, :]
    return out


# speedup vs baseline: 102.3878x; 1.0255x over previous
import functools

import jax
import jax.numpy as jnp
from jax import lax
from jax.experimental import pallas as pl
from jax.experimental.pallas import tpu as pltpu
from jax.experimental.pallas import tpu_sc as plsc

N_NODES = 100000
N_EDGES = 3200000
DIM_IN = 128
BATCH = 4
N_GCN = 3

NC = 2
NS = 16
NPAD = 100096
NPT = NPAD // NS
EPT = N_EDGES // NS
CHUNK = 5000
NCHUNK = EPT // CHUNK
NVEC = NPT // 16


def _proj_body(x_ref, w_ref, b_ref, o_ref):
    x = x_ref[...].astype(jnp.bfloat16).astype(jnp.float32)
    w = w_ref[...].astype(jnp.bfloat16).astype(jnp.float32)
    y = jnp.sum(x * w[0][None, None, :], axis=-1) + b_ref[0, 0]
    o_ref[...] = y[None]


def _project(x, w, bias1):
    BN = 2000
    grid = N_NODES // BN
    out = pl.pallas_call(
        _proj_body,
        grid=(grid,),
        in_specs=[
            pl.BlockSpec((BATCH, BN, DIM_IN), lambda i: (0, i, 0)),
            pl.BlockSpec((1, DIM_IN), lambda i: (0, 0)),
            pl.BlockSpec(memory_space=pltpu.SMEM),
        ],
        out_specs=pl.BlockSpec((1, BATCH, BN), lambda i: (i, 0, 0)),
        out_shape=jax.ShapeDtypeStruct((grid, BATCH, BN), jnp.float32),
    )(x, w, bias1)
    return out.transpose(1, 0, 2).reshape(BATCH, N_NODES)


def _deg_body(dst_hbm, ones_hbm, zeros_hbm, deg_hbm, dg, dstb, onesb, zb,
              sem_i, sem_s):
    c = lax.axis_index("c")
    s = lax.axis_index("s")
    nb = s * NPT
    nsl = pl.ds(nb, NPT)
    eb = (s + c * NS) * (N_EDGES // (2 * NS) // CHUNK) * CHUNK
    ndeg = N_EDGES // (2 * NS) // CHUNK

    pltpu.sync_copy(ones_hbm, onesb)
    pltpu.sync_copy(zeros_hbm, zb)
    pltpu.sync_copy(zb, dg.at[nsl])
    plsc.subcore_barrier()

    def issue_idx(j, p):
        base = eb + j * CHUNK
        pltpu.async_copy(dst_hbm.at[pl.ds(base, CHUNK)], dstb.at[p], sem_i)

    def wait_idx(j, p):
        base = eb + j * CHUNK
        pltpu.make_async_copy(
            dst_hbm.at[pl.ds(base, CHUNK)], dstb.at[p], sem_i).wait()

    def wait_scat(p):
        pltpu.make_async_copy(onesb, dg.at[dstb.at[p]], sem_s).wait()

    issue_idx(0, 0)

    def pair(i, carry):
        for p in (0, 1):
            j = 2 * i + p
            wait_idx(j, p)
            if p == 0:
                @pl.when(i > 0)
                def _():
                    wait_scat(1)
                issue_idx(j + 1, 1)
            else:
                wait_scat(0)

                @pl.when(i < ndeg // 2 - 1)
                def _():
                    issue_idx(j + 1, 0)
            pltpu.async_copy(onesb, dg.at[dstb.at[p]], sem_s, add=True)
        return carry

    lax.fori_loop(0, ndeg // 2, pair, 0)
    wait_scat(1)
    plsc.subcore_barrier()

    pltpu.sync_copy(dg.at[nsl], zb)
    pltpu.sync_copy(zb, deg_hbm.at[c, nsl])


_deg_kernel = functools.partial(
    pl.kernel,
    out_type=jax.ShapeDtypeStruct((NC, NPAD), jnp.float32),
    mesh=plsc.VectorSubcoreMesh(core_axis_name="c", subcore_axis_name="s"),
    compiler_params=pltpu.CompilerParams(
        use_tc_tiling_on_sc=False, needs_layout_passes=False
    ),
    scratch_types=[
        pltpu.VMEM_SHARED((NPAD,), jnp.float32),
        pltpu.VMEM((2, CHUNK), jnp.int32),
        pltpu.VMEM((CHUNK,), jnp.float32),
        pltpu.VMEM((NPT,), jnp.float32),
        pltpu.SemaphoreType.DMA,
        pltpu.SemaphoreType.DMA,
    ],
)(_deg_body)


def _bf16_round(v):
    bits = plsc.bitcast(v, jnp.int32)
    lsb = lax.bitwise_and(lax.shift_right_logical(bits, 16), 1)
    r = lax.bitwise_and(bits + (lsb + 0x7FFF), -65536)
    return plsc.bitcast(r, jnp.float32)


def _sc_body(h0_hbm, src_hbm, dst_hbm, pw_hbm, par_hbm, deg_hbm, zeros_hbm,
             out_hbm,
             t0, t1, s0a, s1a,
             srcb, dstb, gat0, gat1,
             sl0, sl1, dsl, h0b, h1b, zb, pwb, parb, ovec,
             sem_i, sem_g, sem_s):
    c = lax.axis_index("c")
    s = lax.axis_index("s")
    nb = s * NPT
    eb = s * EPT
    nsl = pl.ds(nb, NPT)

    iota = lax.iota(jnp.int32, 16)

    pltpu.sync_copy(par_hbm, parb)
    pltpu.sync_copy(zeros_hbm, zb)
    pltpu.sync_copy(pw_hbm.at[nsl], pwb)
    pltpu.sync_copy(h0_hbm.at[c, 0, nsl], h0b)
    pltpu.sync_copy(h0_hbm.at[c, 1, nsl], h1b)
    pltpu.sync_copy(h0b, t0.at[nsl])
    pltpu.sync_copy(h1b, t1.at[nsl])
    pltpu.sync_copy(zb, s0a.at[nsl])
    pltpu.sync_copy(zb, s1a.at[nsl])
    pltpu.sync_copy(deg_hbm.at[0, nsl], dsl)
    pltpu.sync_copy(deg_hbm.at[1, nsl], sl0)

    def deg_sum(j, carry):
        d16 = pl.ds(j * 16, 16)
        dsl[d16] = dsl[d16] + sl0[d16]
        return carry

    lax.fori_loop(0, NVEC, deg_sum, 0)
    plsc.subcore_barrier()

    accs = (jnp.zeros((16,), jnp.float32), jnp.zeros((16,), jnp.float32))

    for k in range(N_GCN):
        def issue_idx(j, p):
            base = eb + j * CHUNK
            pltpu.async_copy(src_hbm.at[pl.ds(base, CHUNK)], srcb.at[p], sem_i)
            pltpu.async_copy(dst_hbm.at[pl.ds(base, CHUNK)], dstb.at[p], sem_i)

        def wait_idx(j, p):
            base = eb + j * CHUNK
            pltpu.make_async_copy(
                src_hbm.at[pl.ds(base, CHUNK)], srcb.at[p], sem_i).wait()
            pltpu.make_async_copy(
                dst_hbm.at[pl.ds(base, CHUNK)], dstb.at[p], sem_i).wait()

        def issue_scat(p):
            pltpu.async_copy(gat0.at[p], s0a.at[dstb.at[p]], sem_s, add=True)
            pltpu.async_copy(gat1.at[p], s1a.at[dstb.at[p]], sem_s, add=True)

        def wait_scat(p):
            pltpu.make_async_copy(gat0.at[p], s0a.at[dstb.at[p]], sem_s).wait()
            pltpu.make_async_copy(gat1.at[p], s1a.at[dstb.at[p]], sem_s).wait()

        issue_idx(0, 0)

        def edge_pair(i, carry):
            for p in (0, 1):
                j = 2 * i + p
                wait_idx(j, p)
                g0 = pltpu.async_copy(t0.at[srcb.at[p]], gat0.at[p], sem_g)
                g1 = pltpu.async_copy(t1.at[srcb.at[p]], gat1.at[p], sem_g)
                if p == 0:
                    @pl.when(i > 0)
                    def _():
                        wait_scat(1)
                    issue_idx(j + 1, 1)
                else:
                    wait_scat(0)

                    @pl.when(i < NCHUNK // 2 - 1)
                    def _():
                        issue_idx(j + 1, 0)
                g0.wait()
                g1.wait()
                issue_scat(p)
            return carry

        lax.fori_loop(0, NCHUNK // 2, edge_pair, 0)
        wait_scat(1)
        plsc.subcore_barrier()

        pltpu.sync_copy(s0a.at[nsl], sl0)
        pltpu.sync_copy(s1a.at[nsl], sl1)

        ak = parb.at[k][pl.ds(0, 16)]
        bk = parb.at[k][pl.ds(16, 16)]
        ck = parb.at[k][pl.ds(32, 16)]

        def node_step(j, carry):
            a0, a1 = carry
            d16 = pl.ds(j * 16, 16)
            dv = dsl[d16]
            denom = jnp.maximum(dv, 1.0)
            hn0 = (ak * sl0[d16] + bk * dv * h0b[d16]) / denom + ck
            hn0 = jnp.where(hn0 >= 0.0, hn0, 0.1 * hn0)
            h0b[d16] = hn0
            hn1 = (ak * sl1[d16] + bk * dv * h1b[d16]) / denom + ck
            hn1 = jnp.where(hn1 >= 0.0, hn1, 0.1 * hn1)
            h1b[d16] = hn1
            if k == N_GCN - 1:
                pwv = pwb[d16]
                a0 = a0 + _bf16_round(hn0) * pwv
                a1 = a1 + _bf16_round(hn1) * pwv
            return (a0, a1)

        accs = lax.fori_loop(0, NVEC, node_step, accs)

        if k < N_GCN - 1:
            pltpu.sync_copy(zb, s0a.at[nsl])
            pltpu.sync_copy(zb, s1a.at[nsl])
            pltpu.sync_copy(h0b, t0.at[nsl])
            pltpu.sync_copy(h1b, t1.at[nsl])
            plsc.subcore_barrier()

    p0 = jnp.sum(accs[0])
    p1 = jnp.sum(accs[1])
    res = jnp.where(iota == 0, p0, jnp.where(iota == 1, p1, 0.0))
    ovec[...] = res
    pltpu.sync_copy(ovec, out_hbm.at[c, s])


_sc_kernel = functools.partial(
    pl.kernel,
    out_type=jax.ShapeDtypeStruct((NC, NS, 16), jnp.float32),
    mesh=plsc.VectorSubcoreMesh(core_axis_name="c", subcore_axis_name="s"),
    compiler_params=pltpu.CompilerParams(
        use_tc_tiling_on_sc=False, needs_layout_passes=False
    ),
    scratch_types=[
        pltpu.VMEM_SHARED((NPAD,), jnp.float32),
        pltpu.VMEM_SHARED((NPAD,), jnp.float32),
        pltpu.VMEM_SHARED((NPAD,), jnp.float32),
        pltpu.VMEM_SHARED((NPAD,), jnp.float32),
        pltpu.VMEM((2, CHUNK), jnp.int32),
        pltpu.VMEM((2, CHUNK), jnp.int32),
        pltpu.VMEM((2, CHUNK), jnp.float32),
        pltpu.VMEM((2, CHUNK), jnp.float32),
        pltpu.VMEM((NPT,), jnp.float32),
        pltpu.VMEM((NPT,), jnp.float32),
        pltpu.VMEM((NPT,), jnp.float32),
        pltpu.VMEM((NPT,), jnp.float32),
        pltpu.VMEM((NPT,), jnp.float32),
        pltpu.VMEM((NPT,), jnp.float32),
        pltpu.VMEM((NPT,), jnp.float32),
        pltpu.VMEM((N_GCN, 48), jnp.float32),
        pltpu.VMEM((16,), jnp.float32),
        pltpu.SemaphoreType.DMA,
        pltpu.SemaphoreType.DMA,
        pltpu.SemaphoreType.DMA,
    ],
)(_sc_body)


@jax.jit
def kernel(x, edge, edge_weight, parameter, bias1, p1, p2, agg_bias,
           pred_w, pred_b):
    del edge_weight
    w = (jnp.abs(parameter) / DIM_IN).reshape(1, DIM_IN)
    h0 = _project(x, w, bias1)
    h0p = jnp.pad(h0, ((0, 0), (0, NPAD - N_NODES)))
    h0s = h0p.reshape(NC, 2, NPAD)

    src = edge[0]
    dst = edge[1]

    lane = jnp.ones((16,), jnp.float32)
    a3 = jnp.abs(p1[:, 0, 0])
    b3 = jnp.abs(p2[:, 0, 0])
    c3 = agg_bias[:, 0, 0]
    par = jnp.concatenate(
        [a3[:, None] * lane, b3[:, None] * lane, c3[:, None] * lane], axis=1
    )

    pw_r = pred_w[0].astype(jnp.bfloat16).astype(jnp.float32)
    pw = jnp.pad(pw_r, (0, NPAD - N_NODES))
    ones_c = jnp.ones((CHUNK,), jnp.float32)
    zeros_c = jnp.zeros((NPT,), jnp.float32)

    deg = _deg_kernel(dst, ones_c, zeros_c)
    parts = _sc_kernel(h0s, src, dst, pw, par, deg, zeros_c)
    out = parts[:, :, :2].sum(axis=1).reshape(BATCH, 1) + pred_b[None, :]
    return out
